# Initial kernel scaffold; baseline (speedup 1.0000x reference)
import jax
import jax.numpy as jnp
from jax.experimental import pallas as pl


def kernel(x, dct_w, w1, w2, conv_wb):
    raise NotImplementedError("write your pallas kernel here")


# speedup vs baseline: 1.3798x; 1.3798x over previous
import jax
import jax.numpy as jnp
from jax.experimental import pallas as pl
from jax.experimental.pallas import tpu as pltpu


def _gate_pool_kernel(x_ref, dct_ref, w1_ref, w2_ref, att_ref, pool_ref):
    C = x_ref.shape[1]
    x = x_ref[0]

    y = jnp.sum(x * dct_ref[...], axis=1)[None, :]
    h = jnp.maximum(jnp.dot(y, w1_ref[...], preferred_element_type=jnp.float32), 0.0)
    att = jax.nn.sigmoid(jnp.dot(h, w2_ref[...], preferred_element_type=jnp.float32))
    att_ref[0] = att

    mx = jnp.max(x * att[0][:, None], axis=0)
    mn = jnp.dot(att, x, preferred_element_type=jnp.float32)[0] * (1.0 / C)
    pool_ref[0, 0] = mx
    pool_ref[0, 1] = mn


def _conv_kernel(pool_ref, wb_ref, conv_ref):
    H, W = pool_ref.shape[2], pool_ref.shape[3]
    zr = jnp.zeros((3, W), jnp.float32)
    zc = jnp.zeros((H + 6, 3), jnp.float32)
    planes = []
    for c in range(2):
        p = jnp.concatenate([zr, pool_ref[0, c], zr], axis=0)
        planes.append(jnp.concatenate([zc, p, zc], axis=1))

    acc = jnp.zeros((H, W), jnp.float32) + wb_ref[98]
    for c in range(2):
        for dx in range(7):
            col = planes[c][:, dx:dx + W]
            for dy in range(7):
                acc = acc + wb_ref[c * 49 + dy * 7 + dx] * col[dy:dy + H, :]
    conv_ref[0] = acc


def _apply_kernel(x_ref, att_ref, conv_ref, out_ref):
    n = pl.program_id(0)

    cb = conv_ref[...]
    numel = cb.size
    mean = jnp.sum(cb) * (1.0 / numel)
    diff = cb - mean
    var = jnp.sum(diff * diff) * (1.0 / (numel - 1))
    std = jnp.sqrt(var)
    inv_sigma = 1.0 / (jnp.sqrt(2.0 * jnp.pi) * std)

    d = conv_ref[n, :] - mean
    scale = jnp.exp(-(d * d) / (2.0 * var)) * inv_sigma
    gate = att_ref[n, :][:, None] * scale[None, :]
    out_ref[0] = x_ref[0] * gate


def kernel(x, dct_w, w1, w2, conv_wb):
    N, C, H, W = x.shape
    HW = H * W
    Cr = w1.shape[1]

    x2 = x.reshape(N, C, HW)
    dct2 = dct_w.reshape(C, HW)

    att3, pool = pl.pallas_call(
        _gate_pool_kernel,
        grid=(N,),
        in_specs=[
            pl.BlockSpec((1, C, HW), lambda n: (n, 0, 0)),
            pl.BlockSpec((C, HW), lambda n: (0, 0)),
            pl.BlockSpec((C, Cr), lambda n: (0, 0)),
            pl.BlockSpec((Cr, C), lambda n: (0, 0)),
        ],
        out_specs=(
            pl.BlockSpec((1, 1, C), lambda n: (n, 0, 0)),
            pl.BlockSpec((1, 2, HW), lambda n: (n, 0, 0)),
        ),
        out_shape=(
            jax.ShapeDtypeStruct((N, 1, C), jnp.float32),
            jax.ShapeDtypeStruct((N, 2, HW), jnp.float32),
        ),
        compiler_params=pltpu.CompilerParams(dimension_semantics=("parallel",)),
    )(x2, dct2, w1, w2)

    conv = pl.pallas_call(
        _conv_kernel,
        grid=(N,),
        in_specs=[
            pl.BlockSpec((1, 2, H, W), lambda n: (n, 0, 0, 0)),
            pl.BlockSpec(memory_space=pltpu.MemorySpace.SMEM),
        ],
        out_specs=pl.BlockSpec((1, H, W), lambda n: (n, 0, 0)),
        out_shape=jax.ShapeDtypeStruct((N, H, W), jnp.float32),
        compiler_params=pltpu.CompilerParams(dimension_semantics=("parallel",)),
    )(pool.reshape(N, 2, H, W), conv_wb)

    out_flat = pl.pallas_call(
        _apply_kernel,
        grid=(N,),
        in_specs=[
            pl.BlockSpec((1, C, HW), lambda n: (n, 0, 0)),
            pl.BlockSpec((N, C), lambda n: (0, 0)),
            pl.BlockSpec((N, HW), lambda n: (0, 0)),
        ],
        out_specs=pl.BlockSpec((1, C, HW), lambda n: (n, 0, 0)),
        out_shape=jax.ShapeDtypeStruct((N, C, HW), jnp.float32),
        compiler_params=pltpu.CompilerParams(dimension_semantics=("parallel",)),
    )(x2, att3.reshape(N, C), conv.reshape(N, HW))

    return out_flat.reshape(N, C, H, W)
